# Initial kernel scaffold; baseline (speedup 1.0000x reference)
import jax
import jax.numpy as jnp
from jax.experimental import pallas as pl


def kernel(x, x_len, mask, emb_table, W_final):
    raise NotImplementedError("write your pallas kernel here")


# speedup vs baseline: 74.6947x; 74.6947x over previous
import jax
import jax.numpy as jnp
from jax import lax
from jax.experimental import pallas as pl
from jax.experimental.pallas import tpu as pltpu
from jax.experimental.pallas import tpu_sc as plsc

VOCAB = 100000
EMB_D = 100
BATCH = 16384
SEQ_LEN = 100
N_OUT = 50

NC = 2
NS = 16
L = 16
NW = NC * NS

ROWS_PER_BLK = 512
NBLK = -(-VOCAB // ROWS_PER_BLK)
VPAD = NBLK * ROWS_PER_BLK


def _rowmean_body(emb_ref, out_ref):
    i = pl.program_id(0)
    s = jnp.sum(emb_ref[...], axis=1) * (1.0 / EMB_D)
    s2 = s.reshape(1, 1, ROWS_PER_BLK)
    rows = i * ROWS_PER_BLK + lax.broadcasted_iota(
        jnp.int32, (1, 1, ROWS_PER_BLK), 2)
    out_ref[...] = jnp.where(rows == 1, 0.0, s2)


def _rowmean(emb_table):
    return pl.pallas_call(
        _rowmean_body,
        grid=(NBLK,),
        in_specs=[pl.BlockSpec((ROWS_PER_BLK, EMB_D), lambda i: (i, 0))],
        out_specs=pl.BlockSpec((1, 1, ROWS_PER_BLK), lambda i: (i, 0, 0)),
        out_shape=jax.ShapeDtypeStruct((NBLK, 1, ROWS_PER_BLK), jnp.float32),
    )(emb_table)


N_IDX = BATCH * SEQ_LEN
PER_W = N_IDX // NW
CHUNK = 5120
NCHUNK = PER_W // CHUNK


def _gather_body(tab_hbm, idx_hbm, out_hbm, tab_v, idx_v, out_v):
    wid = lax.axis_index("s") * NC + lax.axis_index("c")
    base = wid * PER_W
    pltpu.sync_copy(tab_hbm, tab_v)

    def chunk_body(c, _):
        off = base + c * CHUNK
        pltpu.sync_copy(idx_hbm.at[pl.ds(off, CHUNK)], idx_v)

        def vec_body(i, _):
            iv = idx_v[pl.ds(i * L, L)]
            out_v[pl.ds(i * L, L)] = plsc.load_gather(tab_v, [iv])
            return 0

        lax.fori_loop(0, CHUNK // L, vec_body, 0, unroll=8)
        pltpu.sync_copy(out_v, out_hbm.at[pl.ds(off, CHUNK)])
        return 0

    lax.fori_loop(0, NCHUNK, chunk_body, 0)


_gather = pl.kernel(
    _gather_body,
    out_type=jax.ShapeDtypeStruct((N_IDX,), jnp.float32),
    mesh=plsc.VectorSubcoreMesh(core_axis_name="c", subcore_axis_name="s"),
    compiler_params=pltpu.CompilerParams(needs_layout_passes=False),
    scratch_types=[
        pltpu.VMEM((VPAD,), jnp.float32),
        pltpu.VMEM((CHUNK,), jnp.int32),
        pltpu.VMEM((CHUNK,), jnp.float32),
    ],
)


BM = 1024


def _mm_body(m_ref, w_ref, o_ref):
    o_ref[...] = jnp.dot(m_ref[...], w_ref[...],
                         preferred_element_type=jnp.float32)


def _matmul(m2, w_t):
    return pl.pallas_call(
        _mm_body,
        grid=(BATCH // BM,),
        in_specs=[pl.BlockSpec((BM, SEQ_LEN), lambda i: (i, 0)),
                  pl.BlockSpec((SEQ_LEN, N_OUT), lambda i: (0, 0))],
        out_specs=pl.BlockSpec((BM, N_OUT), lambda i: (i, 0)),
        out_shape=jax.ShapeDtypeStruct((BATCH, N_OUT), jnp.float32),
    )(m2, w_t)


def kernel(x, x_len, mask, emb_table, W_final):
    del x_len, mask
    rowmean = _rowmean(emb_table).reshape(-1)
    m = _gather(rowmean, x.reshape(-1))
    return _matmul(m.reshape(BATCH, SEQ_LEN), W_final.T)


# speedup vs baseline: 112.9047x; 1.5115x over previous
import jax
import jax.numpy as jnp
from jax import lax
from jax.experimental import pallas as pl
from jax.experimental.pallas import tpu as pltpu
from jax.experimental.pallas import tpu_sc as plsc

VOCAB = 100000
EMB_D = 100
BATCH = 16384
SEQ_LEN = 100
N_OUT = 50
LPAD = 128

NC = 2
NS = 16
L = 16
NW = NC * NS

ROWS_PER_BLK = 2048
NBLK = -(-VOCAB // ROWS_PER_BLK)
VPAD = NBLK * ROWS_PER_BLK


def _rowmean_body(emb_ref, out_ref):
    out_ref[...] = jnp.sum(emb_ref[...], axis=1) * (1.0 / EMB_D)


def _rowmean(emb_table):
    return pl.pallas_call(
        _rowmean_body,
        grid=(NBLK,),
        in_specs=[pl.BlockSpec((ROWS_PER_BLK, EMB_D), lambda i: (i, 0))],
        out_specs=pl.BlockSpec((ROWS_PER_BLK,), lambda i: (i,)),
        out_shape=jax.ShapeDtypeStruct((VPAD,), jnp.float32),
    )(emb_table)


ROWS_PER_W = BATCH // NW
CROWS = 64
NCHUNK = ROWS_PER_W // CROWS
ROW_OFFS = (0, 16, 32, 48, 64, 80, 84)


def _gather_body(tab_hbm, x_hbm, out_hbm, tab_v, idx_v, out_v):
    wid = lax.axis_index("s") * NC + lax.axis_index("c")
    base = wid * ROWS_PER_W
    pltpu.sync_copy(tab_hbm, tab_v)
    head = tab_v[pl.ds(0, L)]
    tab_v[pl.ds(0, L)] = jnp.where(
        lax.broadcasted_iota(jnp.int32, (L,), 0) == 1, 0.0, head)

    def chunk_body(c, _):
        r0 = base + c * CROWS
        pltpu.sync_copy(x_hbm.at[pl.ds(r0, CROWS), :], idx_v)

        def row_body(r, _):
            for off in ROW_OFFS:
                iv = idx_v[r, pl.ds(off, L)]
                out_v[r, pl.ds(off, L)] = plsc.load_gather(tab_v, [iv])
            return 0

        lax.fori_loop(0, CROWS, row_body, 0)
        pltpu.sync_copy(out_v, out_hbm.at[pl.ds(r0, CROWS), :])
        return 0

    lax.fori_loop(0, NCHUNK, chunk_body, 0)


_gather = pl.kernel(
    _gather_body,
    out_type=jax.ShapeDtypeStruct((BATCH, LPAD), jnp.float32),
    mesh=plsc.VectorSubcoreMesh(core_axis_name="c", subcore_axis_name="s"),
    compiler_params=pltpu.CompilerParams(needs_layout_passes=False),
    scratch_types=[
        pltpu.VMEM((VPAD,), jnp.float32),
        pltpu.VMEM((CROWS, SEQ_LEN), jnp.int32),
        pltpu.VMEM((CROWS, LPAD), jnp.float32),
    ],
)


BM = 1024


def _mm_body(m_ref, w_ref, o_ref):
    o_ref[...] = jnp.dot(m_ref[...][:, :SEQ_LEN], w_ref[...],
                         preferred_element_type=jnp.float32)


def _matmul(m2, w_t):
    return pl.pallas_call(
        _mm_body,
        grid=(BATCH // BM,),
        in_specs=[pl.BlockSpec((BM, LPAD), lambda i: (i, 0)),
                  pl.BlockSpec((SEQ_LEN, N_OUT), lambda i: (0, 0))],
        out_specs=pl.BlockSpec((BM, N_OUT), lambda i: (i, 0)),
        out_shape=jax.ShapeDtypeStruct((BATCH, N_OUT), jnp.float32),
    )(m2, w_t)


def kernel(x, x_len, mask, emb_table, W_final):
    del x_len, mask
    rowmean = _rowmean(emb_table)
    m = _gather(rowmean, x)
    return _matmul(m, W_final.T)


# speedup vs baseline: 112.9090x; 1.0000x over previous
import jax
import jax.numpy as jnp
from jax import lax
from jax.experimental import pallas as pl
from jax.experimental.pallas import tpu as pltpu
from jax.experimental.pallas import tpu_sc as plsc

VOCAB = 100000
EMB_D = 100
BATCH = 16384
SEQ_LEN = 100
N_OUT = 50
LPAD = 128

NC = 2
NS = 16
L = 16
NW = NC * NS

ROWS_PER_BLK = 2048
NBLK = -(-VOCAB // ROWS_PER_BLK)
VPAD = NBLK * ROWS_PER_BLK


def _rowmean_body(emb_ref, out_ref):
    out_ref[...] = jnp.sum(emb_ref[...], axis=1) * (1.0 / EMB_D)


def _rowmean(emb_table):
    return pl.pallas_call(
        _rowmean_body,
        grid=(NBLK,),
        in_specs=[pl.BlockSpec((ROWS_PER_BLK, EMB_D), lambda i: (i, 0))],
        out_specs=pl.BlockSpec((ROWS_PER_BLK,), lambda i: (i,)),
        out_shape=jax.ShapeDtypeStruct((VPAD,), jnp.float32),
    )(emb_table)


ROWS_PER_W = BATCH // NW
CROWS = 64
NCHUNK = ROWS_PER_W // CROWS
ROW_OFFS = (0, 16, 32, 48, 64, 80, 84)


def _gather_body(tab_hbm, x_hbm, out_hbm, tab_v, idx_v, out_v):
    wid = lax.axis_index("s") * NC + lax.axis_index("c")
    base = wid * ROWS_PER_W
    pltpu.sync_copy(tab_hbm, tab_v)
    head = tab_v[pl.ds(0, L)]
    tab_v[pl.ds(0, L)] = jnp.where(
        lax.broadcasted_iota(jnp.int32, (L,), 0) == 1, 0.0, head)

    def chunk_body(c, _):
        r0 = base + c * CROWS
        pltpu.sync_copy(x_hbm.at[pl.ds(r0, CROWS), :], idx_v)

        def row_body(r, _):
            for off in ROW_OFFS:
                iv = idx_v[r, pl.ds(off, L)]
                out_v[r, pl.ds(off, L)] = plsc.load_gather(tab_v, [iv])
            return 0

        lax.fori_loop(0, CROWS, row_body, 0)
        pltpu.sync_copy(out_v, out_hbm.at[pl.ds(r0, CROWS), :])
        return 0

    lax.fori_loop(0, NCHUNK, chunk_body, 0)


_gather = pl.kernel(
    _gather_body,
    out_type=jax.ShapeDtypeStruct((BATCH, LPAD), jnp.float32),
    mesh=plsc.VectorSubcoreMesh(core_axis_name="c", subcore_axis_name="s"),
    compiler_params=pltpu.CompilerParams(needs_layout_passes=False,
                                         use_tc_tiling_on_sc=True),
    scratch_types=[
        pltpu.VMEM((VPAD,), jnp.float32),
        pltpu.VMEM((CROWS, SEQ_LEN), jnp.int32),
        pltpu.VMEM((CROWS, LPAD), jnp.float32),
    ],
)


BM = 1024


def _mm_body(m_ref, w_ref, o_ref):
    o_ref[...] = jnp.dot(m_ref[...][:, :SEQ_LEN], w_ref[...],
                         preferred_element_type=jnp.float32)


def _matmul(m2, w_t):
    return pl.pallas_call(
        _mm_body,
        grid=(BATCH // BM,),
        in_specs=[pl.BlockSpec((BM, LPAD), lambda i: (i, 0)),
                  pl.BlockSpec((SEQ_LEN, N_OUT), lambda i: (0, 0))],
        out_specs=pl.BlockSpec((BM, N_OUT), lambda i: (i, 0)),
        out_shape=jax.ShapeDtypeStruct((BATCH, N_OUT), jnp.float32),
    )(m2, w_t)


def kernel(x, x_len, mask, emb_table, W_final):
    del x_len, mask
    rowmean = _rowmean(emb_table)
    m = _gather(rowmean, x)
    return _matmul(m, W_final.T)


# speedup vs baseline: 113.0486x; 1.0012x over previous
import jax
import jax.numpy as jnp
from jax import lax
from jax.experimental import pallas as pl
from jax.experimental.pallas import tpu as pltpu
from jax.experimental.pallas import tpu_sc as plsc

VOCAB = 100000
EMB_D = 100
BATCH = 16384
SEQ_LEN = 100
N_OUT = 50
LPAD = 128

NC = 2
NS = 16
L = 16
NW = NC * NS

ROWS_PER_BLK = 512
NSTRIPE = 4
ROWS_PER_STEP = ROWS_PER_BLK * NSTRIPE
NSTEP = -(-VOCAB // ROWS_PER_STEP)
VPAD = NSTEP * ROWS_PER_STEP


def _rowmean_body(*refs):
    emb_refs, out_ref = refs[:NSTRIPE], refs[NSTRIPE]
    for s in range(NSTRIPE):
        out_ref[pl.ds(s * ROWS_PER_BLK, ROWS_PER_BLK)] = (
            jnp.sum(emb_refs[s][...], axis=1) * (1.0 / EMB_D))


def _rowmean(emb_table):
    return pl.pallas_call(
        _rowmean_body,
        grid=(NSTEP,),
        in_specs=[
            pl.BlockSpec((ROWS_PER_BLK, EMB_D),
                         lambda i, s=s: (i * NSTRIPE + s, 0))
            for s in range(NSTRIPE)
        ],
        out_specs=pl.BlockSpec((ROWS_PER_STEP,), lambda i: (i,)),
        out_shape=jax.ShapeDtypeStruct((VPAD,), jnp.float32),
    )(*([emb_table] * NSTRIPE))


ROWS_PER_W = BATCH // NW
CROWS = 64
NCHUNK = ROWS_PER_W // CROWS
ROW_OFFS = (0, 16, 32, 48, 64, 80, 84)


def _gather_body(tab_hbm, x_hbm, out_hbm, tab_v, idx_v, out_v):
    wid = lax.axis_index("s") * NC + lax.axis_index("c")
    base = wid * ROWS_PER_W
    pltpu.sync_copy(tab_hbm, tab_v)
    head = tab_v[pl.ds(0, L)]
    tab_v[pl.ds(0, L)] = jnp.where(
        lax.broadcasted_iota(jnp.int32, (L,), 0) == 1, 0.0, head)

    def chunk_body(c, _):
        r0 = base + c * CROWS
        pltpu.sync_copy(x_hbm.at[pl.ds(r0, CROWS), :], idx_v)

        def row_body(r, _):
            for off in ROW_OFFS:
                iv = idx_v[r, pl.ds(off, L)]
                out_v[r, pl.ds(off, L)] = plsc.load_gather(tab_v, [iv])
            return 0

        lax.fori_loop(0, CROWS, row_body, 0)
        pltpu.sync_copy(out_v, out_hbm.at[pl.ds(r0, CROWS), :])
        return 0

    lax.fori_loop(0, NCHUNK, chunk_body, 0)


_gather = pl.kernel(
    _gather_body,
    out_type=jax.ShapeDtypeStruct((BATCH, LPAD), jnp.float32),
    mesh=plsc.VectorSubcoreMesh(core_axis_name="c", subcore_axis_name="s"),
    compiler_params=pltpu.CompilerParams(needs_layout_passes=False,
                                         use_tc_tiling_on_sc=True),
    scratch_types=[
        pltpu.VMEM((VPAD,), jnp.float32),
        pltpu.VMEM((CROWS, SEQ_LEN), jnp.int32),
        pltpu.VMEM((CROWS, LPAD), jnp.float32),
    ],
)


BM = 1024


def _mm_body(m_ref, w_ref, o_ref):
    o_ref[...] = jnp.dot(m_ref[...][:, :SEQ_LEN], w_ref[...],
                         preferred_element_type=jnp.float32)


def _matmul(m2, w_t):
    return pl.pallas_call(
        _mm_body,
        grid=(BATCH // BM,),
        in_specs=[pl.BlockSpec((BM, LPAD), lambda i: (i, 0)),
                  pl.BlockSpec((SEQ_LEN, N_OUT), lambda i: (0, 0))],
        out_specs=pl.BlockSpec((BM, N_OUT), lambda i: (i, 0)),
        out_shape=jax.ShapeDtypeStruct((BATCH, N_OUT), jnp.float32),
    )(m2, w_t)


def kernel(x, x_len, mask, emb_table, W_final):
    del x_len, mask
    rowmean = _rowmean(emb_table)
    m = _gather(rowmean, x)
    return _matmul(m, W_final.T)


# speedup vs baseline: 126.4618x; 1.1186x over previous
import jax
import jax.numpy as jnp
from jax import lax
from jax.experimental import pallas as pl
from jax.experimental.pallas import tpu as pltpu
from jax.experimental.pallas import tpu_sc as plsc

VOCAB = 100000
EMB_D = 100
BATCH = 16384
SEQ_LEN = 100
N_OUT = 50
LPAD = 128

NC = 2
NS = 16
L = 16
NW = NC * NS

ROWS_PER_BLK = 512
NSTRIPE = 4
ROWS_PER_STEP = ROWS_PER_BLK * NSTRIPE
NSTEP = -(-VOCAB // ROWS_PER_STEP)
VPAD = NSTEP * ROWS_PER_STEP


def _rowmean_body(*refs):
    emb_refs, out_ref = refs[:NSTRIPE], refs[NSTRIPE]
    ones_row = jnp.full((1, EMB_D), 1.0 / EMB_D, jnp.float32)
    pieces = []
    for s in range(NSTRIPE):
        pieces.append(lax.dot_general(
            ones_row, emb_refs[s][...],
            (((1,), (1,)), ((), ())),
            preferred_element_type=jnp.float32))
    out_ref[...] = jnp.concatenate(pieces, axis=1).reshape(ROWS_PER_STEP)


def _rowmean(emb_table):
    return pl.pallas_call(
        _rowmean_body,
        grid=(NSTEP,),
        in_specs=[
            pl.BlockSpec((ROWS_PER_BLK, EMB_D),
                         lambda i, s=s: (i * NSTRIPE + s, 0))
            for s in range(NSTRIPE)
        ],
        out_specs=pl.BlockSpec((ROWS_PER_STEP,), lambda i: (i,)),
        out_shape=jax.ShapeDtypeStruct((VPAD,), jnp.float32),
    )(*([emb_table] * NSTRIPE))


ROWS_PER_W = BATCH // NW
CROWS = 64
NCHUNK = ROWS_PER_W // CROWS
ROW_OFFS = (0, 16, 32, 48, 64, 80, 84)


def _gather_body(tab_hbm, x_hbm, out_hbm, tab_v, idx_v, out_v):
    wid = lax.axis_index("s") * NC + lax.axis_index("c")
    base = wid * ROWS_PER_W
    pltpu.sync_copy(tab_hbm, tab_v)
    head = tab_v[pl.ds(0, L)]
    tab_v[pl.ds(0, L)] = jnp.where(
        lax.broadcasted_iota(jnp.int32, (L,), 0) == 1, 0.0, head)

    def chunk_body(c, _):
        r0 = base + c * CROWS
        pltpu.sync_copy(x_hbm.at[pl.ds(r0, CROWS), :], idx_v)

        def row_body(r, _):
            for off in ROW_OFFS:
                iv = idx_v[r, pl.ds(off, L)]
                out_v[r, pl.ds(off, L)] = plsc.load_gather(tab_v, [iv])
            return 0

        lax.fori_loop(0, CROWS, row_body, 0)
        pltpu.sync_copy(out_v, out_hbm.at[pl.ds(r0, CROWS), :])
        return 0

    lax.fori_loop(0, NCHUNK, chunk_body, 0)


_gather = pl.kernel(
    _gather_body,
    out_type=jax.ShapeDtypeStruct((BATCH, LPAD), jnp.float32),
    mesh=plsc.VectorSubcoreMesh(core_axis_name="c", subcore_axis_name="s"),
    compiler_params=pltpu.CompilerParams(needs_layout_passes=False,
                                         use_tc_tiling_on_sc=True),
    scratch_types=[
        pltpu.VMEM((VPAD,), jnp.float32),
        pltpu.VMEM((CROWS, SEQ_LEN), jnp.int32),
        pltpu.VMEM((CROWS, LPAD), jnp.float32),
    ],
)


BM = 1024


def _mm_body(m_ref, w_ref, o_ref):
    o_ref[...] = jnp.dot(m_ref[...][:, :SEQ_LEN], w_ref[...],
                         preferred_element_type=jnp.float32)


def _matmul(m2, w_t):
    return pl.pallas_call(
        _mm_body,
        grid=(BATCH // BM,),
        in_specs=[pl.BlockSpec((BM, LPAD), lambda i: (i, 0)),
                  pl.BlockSpec((SEQ_LEN, N_OUT), lambda i: (0, 0))],
        out_specs=pl.BlockSpec((BM, N_OUT), lambda i: (i, 0)),
        out_shape=jax.ShapeDtypeStruct((BATCH, N_OUT), jnp.float32),
    )(m2, w_t)


def kernel(x, x_len, mask, emb_table, W_final):
    del x_len, mask
    rowmean = _rowmean(emb_table)
    m = _gather(rowmean, x)
    return _matmul(m, W_final.T)
